# baseline (speedup 1.0000x reference)
import jax
import jax.numpy as jnp
from jax import lax
from jax.experimental import pallas as pl
from jax.experimental.pallas import tpu as pltpu

_B, _C, _H, _W = 64, 3, 224, 224
_P, _GH, _GW, _D = 14, 16, 16, 768
_MAXT = 10
_NP = _GH * _GW
_K = _C * _P * _P
_PATCH_H = _H / _GH
_PATCH_W = _W / _GW
_BB = 8


def _fused_body(xt_ref, boxes_ref, wp_ref, wa_ref, cls_ref, obj_ref, attn_ref):
    wp = wp_ref[...]
    wa = wa_ref[...]
    wc = jnp.dot(wp, wa, preferred_element_type=jnp.float32)
    wc = wc.astype(jnp.bfloat16)

    for i in range(_BB):
        xt = xt_ref[i]
        bx = boxes_ref[i]

        logits = lax.dot_general(wc, xt, (((0,), (1,)), ((), ())),
                                 preferred_element_type=jnp.float32)

        x0 = jnp.clip(bx[:, 0] * _W, 0.0, float(_W))
        y0 = jnp.clip(bx[:, 1] * _H, 0.0, float(_H))
        x1 = jnp.clip(bx[:, 2] * _W, 0.0, float(_W))
        y1 = jnp.clip(bx[:, 3] * _H, 0.0, float(_H))
        x0i = jnp.clip(jnp.floor(x0 / _PATCH_W).astype(jnp.int32), 0, _GW - 1)
        y0i = jnp.clip(jnp.floor(y0 / _PATCH_H).astype(jnp.int32), 0, _GH - 1)
        x1i = jnp.clip(jnp.ceil(x1 / _PATCH_W).astype(jnp.int32), x0i + 1, _GW)
        y1i = jnp.clip(jnp.ceil(y1 / _PATCH_H).astype(jnp.int32), y0i + 1, _GH)

        p_ids = lax.broadcasted_iota(jnp.int32, (_MAXT, _NP), 1)
        gy = p_ids // _GW
        gx = p_ids % _GW
        mask = ((gy >= y0i[:, None]) & (gy < y1i[:, None]) &
                (gx >= x0i[:, None]) & (gx < x1i[:, None]))

        neg = jnp.float32(-1e30)
        ml = jnp.where(mask, logits, neg)
        ml = ml - jnp.max(ml, axis=-1, keepdims=True)
        ew = jnp.exp(ml)
        ew = jnp.where(mask, ew, 0.0)
        w = ew / jnp.sum(ew, axis=-1, keepdims=True)

        vcat = jnp.concatenate(
            [w, jnp.full((1, _NP), 1.0 / _NP, jnp.float32)], 0)
        vcat16 = vcat.astype(jnp.bfloat16)
        zcat = jnp.dot(vcat16, xt, preferred_element_type=jnp.float32)
        zcat16 = zcat.astype(jnp.bfloat16)
        out11 = jnp.dot(zcat16, wp, preferred_element_type=jnp.float32)

        obj_ref[i] = out11[:_MAXT]
        cls_ref[i] = out11[_MAXT:]
        attn_ref[i] = w


def _half(img_half, boxes_half, wp16, wa16, hb):
    img6 = img_half.reshape(hb, _C, _GH, _P, _GW, _P)
    xt = lax.reshape(img6, (hb, _NP, _K), dimensions=(0, 2, 4, 1, 3, 5))
    nb = hb // _BB
    return pl.pallas_call(
        _fused_body,
        grid=(nb,),
        in_specs=[
            pl.BlockSpec((_BB, _NP, _K), lambda b: (b, 0, 0)),
            pl.BlockSpec((_BB, _MAXT, 4), lambda b: (b, 0, 0)),
            pl.BlockSpec((_K, _D), lambda b: (0, 0)),
            pl.BlockSpec((_D, 1), lambda b: (0, 0)),
        ],
        out_specs=[
            pl.BlockSpec((_BB, 1, _D), lambda b: (b, 0, 0)),
            pl.BlockSpec((_BB, _MAXT, _D), lambda b: (b, 0, 0)),
            pl.BlockSpec((_BB, _MAXT, _NP), lambda b: (b, 0, 0)),
        ],
        out_shape=[
            jax.ShapeDtypeStruct((hb, 1, _D), jnp.float32),
            jax.ShapeDtypeStruct((hb, _MAXT, _D), jnp.float32),
            jax.ShapeDtypeStruct((hb, _MAXT, _NP), jnp.float32),
        ],
    )(xt, boxes_half, wp16, wa16)


def kernel(images, boxes, scores, W_patch, W_att, b_att):
    img_bf = images.astype(jnp.bfloat16)
    wp16 = W_patch.astype(jnp.bfloat16)
    wa16 = W_att.astype(jnp.bfloat16)

    h = _B // 2
    cls1, obj1, attn1 = _half(img_bf[:h], boxes[:h], wp16, wa16, h)
    cls2, obj2, attn2 = _half(img_bf[h:], boxes[h:], wp16, wa16, h)
    cls_tokens = jnp.concatenate([cls1, cls2], 0)
    object_tokens = jnp.concatenate([obj1, obj2], 0)
    attention_maps = jnp.concatenate([attn1, attn2], 0)

    object_mask = jnp.ones((_B, _MAXT), dtype=bool)
    return (cls_tokens.reshape(_B, _D), object_tokens, object_mask, boxes,
            scores, attention_maps)


# speedup vs baseline: 1.4149x; 1.4149x over previous
import jax
import jax.numpy as jnp
from jax import lax
from jax.experimental import pallas as pl
from jax.experimental.pallas import tpu as pltpu

_B, _C, _H, _W = 64, 3, 224, 224
_P, _GH, _GW, _D = 14, 16, 16, 768
_MAXT = 10
_NP = _GH * _GW
_K = _C * _P * _P
_PATCH_H = _H / _GH
_PATCH_W = _W / _GW
_BB = 8


def _fused_body(xt_ref, boxes_ref, wp_ref, wa_ref, cls_ref, obj_ref, attn_ref):
    wp = wp_ref[...]
    wa = wa_ref[...]
    wc = jnp.dot(wp, wa, preferred_element_type=jnp.float32)
    wc = wc.astype(jnp.bfloat16)

    for i in range(_BB):
        xt = xt_ref[i]
        bx = boxes_ref[i]

        logits = lax.dot_general(wc, xt, (((0,), (1,)), ((), ())),
                                 preferred_element_type=jnp.float32)

        x0 = jnp.clip(bx[:, 0] * _W, 0.0, float(_W))
        y0 = jnp.clip(bx[:, 1] * _H, 0.0, float(_H))
        x1 = jnp.clip(bx[:, 2] * _W, 0.0, float(_W))
        y1 = jnp.clip(bx[:, 3] * _H, 0.0, float(_H))
        x0i = jnp.clip(jnp.floor(x0 / _PATCH_W).astype(jnp.int32), 0, _GW - 1)
        y0i = jnp.clip(jnp.floor(y0 / _PATCH_H).astype(jnp.int32), 0, _GH - 1)
        x1i = jnp.clip(jnp.ceil(x1 / _PATCH_W).astype(jnp.int32), x0i + 1, _GW)
        y1i = jnp.clip(jnp.ceil(y1 / _PATCH_H).astype(jnp.int32), y0i + 1, _GH)

        p_ids = lax.broadcasted_iota(jnp.int32, (_MAXT, _NP), 1)
        gy = p_ids // _GW
        gx = p_ids % _GW
        mask = ((gy >= y0i[:, None]) & (gy < y1i[:, None]) &
                (gx >= x0i[:, None]) & (gx < x1i[:, None]))

        neg = jnp.float32(-1e30)
        ml = jnp.where(mask, logits, neg)
        ml = ml - jnp.max(ml, axis=-1, keepdims=True)
        ew = jnp.exp(ml)
        ew = jnp.where(mask, ew, 0.0)
        w = ew / jnp.sum(ew, axis=-1, keepdims=True)

        vcat = jnp.concatenate(
            [w, jnp.full((1, _NP), 1.0 / _NP, jnp.float32)], 0)
        vcat16 = vcat.astype(jnp.bfloat16)
        zcat = jnp.dot(vcat16, xt, preferred_element_type=jnp.float32)
        zcat16 = zcat.astype(jnp.bfloat16)
        out11 = jnp.dot(zcat16, wp, preferred_element_type=jnp.float32)

        obj_ref[i] = out11[:_MAXT]
        cls_ref[i] = out11[_MAXT:]
        attn_ref[i] = w


def kernel(images, boxes, scores, W_patch, W_att, b_att):
    img6 = images.astype(jnp.bfloat16).reshape(_B, _C, _GH, _P, _GW, _P)
    xt = lax.reshape(img6, (_B, _NP, _K), dimensions=(0, 2, 4, 1, 3, 5))
    wp16 = W_patch.astype(jnp.bfloat16)
    wa16 = W_att.astype(jnp.bfloat16)

    nb = _B // _BB
    cls_tokens, object_tokens, attention_maps = pl.pallas_call(
        _fused_body,
        grid=(nb,),
        in_specs=[
            pl.BlockSpec((_BB, _NP, _K), lambda b: (b, 0, 0)),
            pl.BlockSpec((_BB, _MAXT, 4), lambda b: (b, 0, 0)),
            pl.BlockSpec((_K, _D), lambda b: (0, 0)),
            pl.BlockSpec((_D, 1), lambda b: (0, 0)),
        ],
        out_specs=[
            pl.BlockSpec((_BB, 1, _D), lambda b: (b, 0, 0)),
            pl.BlockSpec((_BB, _MAXT, _D), lambda b: (b, 0, 0)),
            pl.BlockSpec((_BB, _MAXT, _NP), lambda b: (b, 0, 0)),
        ],
        out_shape=[
            jax.ShapeDtypeStruct((_B, 1, _D), jnp.float32),
            jax.ShapeDtypeStruct((_B, _MAXT, _D), jnp.float32),
            jax.ShapeDtypeStruct((_B, _MAXT, _NP), jnp.float32),
        ],
    )(xt, boxes, wp16, wa16)

    object_mask = jnp.ones((_B, _MAXT), dtype=bool)
    return (cls_tokens.reshape(_B, _D), object_tokens, object_mask, boxes,
            scores, attention_maps)
